# baseline (speedup 1.0000x reference)
import functools

import jax
import jax.numpy as jnp
from jax import lax
from jax.experimental import pallas as pl
from jax.experimental.pallas import tpu as pltpu
from jax.experimental.pallas import tpu_sc as plsc

D = 64
NC = 2
NS = 16
NW = NC * NS
C = 128
NB = 5


@functools.lru_cache(maxsize=None)
def _build(total):
    per_w = total // NW
    G = per_w // C
    assert per_w % C == 0 and G % NB == 0 and G > NB

    mesh = plsc.VectorSubcoreMesh(core_axis_name="c", subcore_axis_name="s")

    @functools.partial(
        pl.kernel,
        mesh=mesh,
        out_type=jax.ShapeDtypeStruct((total // 2, 2 * D), jnp.float32),
        scratch_types=(
            [
                pltpu.VMEM((G, C), jnp.int32),
                pltpu.VMEM((G, C), jnp.int32),
                pltpu.VMEM((C // 2, 2 * D), jnp.float32),
            ]
            + [pltpu.VMEM((C, 2 * D), jnp.float32) for _ in range(NB)]
            + [pltpu.SemaphoreType.DMA for _ in range(NB)]
        ),
    )
    def k(table_hbm, pair_hbm, off_hbm, out_hbm, idx_v, off_v, out_v, *rest):
        bufs = rest[:NB]
        sems = rest[NB:]
        wid = lax.axis_index("s") * NC + lax.axis_index("c")
        base = wid * (per_w // 2)

        pltpu.sync_copy(pair_hbm.at[wid], idx_v)
        pltpu.sync_copy(off_hbm.at[wid], off_v)

        for b in range(NB):
            pltpu.async_copy(table_hbm.at[idx_v.at[b]], bufs[b], sems[b])

        def chunk(g, buf):
            def group(grp, carry):
                offs = off_v[g, pl.ds(grp * 16, 16)]
                for l in range(16):
                    off = offs[l]
                    for q in range(4):
                        out_v[grp * 8 + l // 2, pl.ds((l % 2) * D + 16 * q, 16)] = (
                            buf[grp * 16 + l, pl.ds(off + 16 * q, 16)]
                        )
                return carry

            lax.fori_loop(0, C // 16, group, 0)
            pltpu.sync_copy(out_v, out_hbm.at[pl.ds(base + g * (C // 2), C // 2)])

        def outer(o, carry):
            for b in range(NB):
                g = o * NB + b
                pltpu.make_async_copy(
                    table_hbm.at[idx_v.at[g]], bufs[b], sems[b]
                ).wait()
                chunk(g, bufs[b])
                pltpu.async_copy(
                    table_hbm.at[idx_v.at[g + NB]], bufs[b], sems[b]
                )
            return carry

        lax.fori_loop(0, (G - NB) // NB, outer, 0)

        for b in range(NB):
            g = G - NB + b
            pltpu.make_async_copy(
                table_hbm.at[idx_v.at[g]], bufs[b], sems[b]
            ).wait()
            chunk(g, bufs[b])

    return k


def kernel(inputs, embedding):
    bsz, hist = inputs.shape
    total = bsz * hist
    nv, d = embedding.shape
    idx = inputs.reshape(NW, total // (NW * C), C).astype(jnp.int32)
    table2 = embedding.reshape(nv // 2, 2 * d)
    out = _build(total)(table2, idx >> 1, (idx & 1) * d)
    return out.reshape(bsz, hist, d)


# speedup vs baseline: 1.0003x; 1.0003x over previous
import functools

import jax
import jax.numpy as jnp
from jax import lax
from jax.experimental import pallas as pl
from jax.experimental.pallas import tpu as pltpu
from jax.experimental.pallas import tpu_sc as plsc

D = 64
NC = 2
NS = 16
NW = NC * NS
C = 128
NB = 5


@functools.lru_cache(maxsize=None)
def _build(total):
    per_w = total // NW
    G = per_w // C
    assert per_w % C == 0 and G % NB == 0 and G > NB

    mesh = plsc.VectorSubcoreMesh(core_axis_name="c", subcore_axis_name="s")

    @functools.partial(
        pl.kernel,
        mesh=mesh,
        out_type=jax.ShapeDtypeStruct((total // 2, 2 * D), jnp.float32),
        scratch_types=(
            [
                pltpu.VMEM((G, C), jnp.int32),
                pltpu.VMEM((G, C), jnp.int32),
                pltpu.VMEM((C // 2, 2 * D), jnp.float32),
            ]
            + [pltpu.VMEM((C, 2 * D), jnp.float32) for _ in range(NB)]
            + [pltpu.SemaphoreType.DMA for _ in range(NB)]
        ),
        compiler_params=pltpu.CompilerParams(use_tc_tiling_on_sc=True),
    )
    def k(table_hbm, pair_hbm, off_hbm, out_hbm, idx_v, off_v, out_v, *rest):
        bufs = rest[:NB]
        sems = rest[NB:]
        wid = lax.axis_index("s") * NC + lax.axis_index("c")
        base = wid * (per_w // 2)

        pltpu.sync_copy(pair_hbm.at[wid], idx_v)
        pltpu.sync_copy(off_hbm.at[wid], off_v)

        for b in range(NB):
            pltpu.async_copy(table_hbm.at[idx_v.at[b]], bufs[b], sems[b])

        def chunk(g, buf):
            def group(grp, carry):
                offs = off_v[g, pl.ds(grp * 16, 16)]
                for l in range(16):
                    off = offs[l]
                    for q in range(4):
                        out_v[grp * 8 + l // 2, pl.ds((l % 2) * D + 16 * q, 16)] = (
                            buf[grp * 16 + l, pl.ds(off + 16 * q, 16)]
                        )
                return carry

            lax.fori_loop(0, C // 16, group, 0)
            pltpu.sync_copy(out_v, out_hbm.at[pl.ds(base + g * (C // 2), C // 2)])

        def outer(o, carry):
            for b in range(NB):
                g = o * NB + b
                pltpu.make_async_copy(
                    table_hbm.at[idx_v.at[g]], bufs[b], sems[b]
                ).wait()
                chunk(g, bufs[b])
                pltpu.async_copy(
                    table_hbm.at[idx_v.at[g + NB]], bufs[b], sems[b]
                )
            return carry

        lax.fori_loop(0, (G - NB) // NB, outer, 0)

        for b in range(NB):
            g = G - NB + b
            pltpu.make_async_copy(
                table_hbm.at[idx_v.at[g]], bufs[b], sems[b]
            ).wait()
            chunk(g, bufs[b])

    return k


def kernel(inputs, embedding):
    bsz, hist = inputs.shape
    total = bsz * hist
    nv, d = embedding.shape
    idx = inputs.reshape(NW, total // (NW * C), C).astype(jnp.int32)
    table2 = embedding.reshape(nv // 2, 2 * d)
    out = _build(total)(table2, idx >> 1, (idx & 1) * d)
    return out.reshape(bsz, hist, d)


# speedup vs baseline: 1.1255x; 1.1251x over previous
import functools

import jax
import jax.numpy as jnp
from jax import lax
from jax.experimental import pallas as pl
from jax.experimental.pallas import tpu as pltpu
from jax.experimental.pallas import tpu_sc as plsc

D = 64
NC = 2
NS = 16
NW = NC * NS
C = 128
NB = 5


@functools.lru_cache(maxsize=None)
def _build(bsz, hist, nv):
    assert bsz == NW * C and hist % NB == 0

    mesh = plsc.VectorSubcoreMesh(core_axis_name="c", subcore_axis_name="s")

    @functools.partial(
        pl.kernel,
        mesh=mesh,
        out_type=jax.ShapeDtypeStruct((bsz, hist, D), jnp.float32),
        scratch_types=(
            [pltpu.VMEM((hist, C), jnp.int32)]
            + [pltpu.VMEM((C, D), jnp.float32) for _ in range(NB)]
            + [pltpu.SemaphoreType.DMA for _ in range(NB)]
        ),
        compiler_params=pltpu.CompilerParams(use_tc_tiling_on_sc=False),
    )
    def k(table_hbm, idxt_hbm, out_hbm, idx_v, *rest):
        bufs = rest[:NB]
        sems = rest[NB:]
        wid = lax.axis_index("s") * NC + lax.axis_index("c")
        base = wid * C

        pltpu.sync_copy(idxt_hbm.at[:, pl.ds(base, C)], idx_v)

        for b in range(NB):
            pltpu.async_copy(table_hbm.at[idx_v.at[b]], bufs[b], sems[b])

        def outer(o, carry):
            for b in range(NB):
                h = o * NB + b
                pltpu.make_async_copy(
                    table_hbm.at[idx_v.at[h]], bufs[b], sems[b]
                ).wait()
                pltpu.sync_copy(
                    bufs[b], out_hbm.at[pl.ds(base, C), h]
                )
                pltpu.async_copy(
                    table_hbm.at[idx_v.at[h + NB]], bufs[b], sems[b]
                )
            return carry

        lax.fori_loop(0, (hist - NB) // NB, outer, 0)

        for b in range(NB):
            h = hist - NB + b
            pltpu.make_async_copy(
                table_hbm.at[idx_v.at[h]], bufs[b], sems[b]
            ).wait()
            pltpu.sync_copy(
                bufs[b], out_hbm.at[pl.ds(base, C), h]
            )

    return k


def kernel(inputs, embedding):
    bsz, hist = inputs.shape
    nv, d = embedding.shape
    idx_t = inputs.T.astype(jnp.int32)
    return _build(bsz, hist, nv)(embedding, idx_t)
